# Initial kernel scaffold; baseline (speedup 1.0000x reference)
import functools

import jax
import jax.numpy as jnp
from jax import lax
from jax.experimental import pallas as pl
from jax.experimental.pallas import tpu as pltpu
from jax.experimental.pallas import tpu_sc as plsc

N = 10000
D = 256
H = 128
E = 160000
NC = 2
NS = 16
CHUNK = 128
E_PAD = 163840
EPT = E_PAD // NS
NCHUNKS = EPT // CHUNK
EPW = E_PAD // (NS * NC)
ROWS_PER_TILE = 632
ACC_ROWS = ROWS_PER_TILE * NS
NBLK = 10
BLK = N // NBLK

_mesh = plsc.VectorSubcoreMesh(core_axis_name="c", subcore_axis_name="s")


@functools.partial(
    pl.kernel,
    out_type=jax.ShapeDtypeStruct((NC, ACC_ROWS, 16), jnp.float32),
    mesh=_mesh,
    scratch_types=[
        pltpu.VMEM((CHUNK,), jnp.int32),
        pltpu.VMEM((CHUNK, 16), jnp.float32),
        pltpu.VMEM_SHARED((ACC_ROWS, 16), jnp.float32),
    ],
)
def _deg_kernel(dst_hbm, zeros_hbm, ones_hbm, out_hbm, dst_v, ones_v, acc_sh):
    c = lax.axis_index("c")
    s = lax.axis_index("s")
    wid = s * NC + c
    r0 = s * ROWS_PER_TILE
    pltpu.sync_copy(zeros_hbm.at[pl.ds(r0, ROWS_PER_TILE)],
                    acc_sh.at[pl.ds(r0, ROWS_PER_TILE)])
    pltpu.sync_copy(ones_hbm, ones_v)
    plsc.subcore_barrier()

    def body(i, carry):
        base = wid * EPW + i * CHUNK
        pltpu.sync_copy(dst_hbm.at[pl.ds(base, CHUNK)], dst_v)
        pltpu.sync_copy(ones_v, acc_sh.at[dst_v], add=True)
        return carry

    lax.fori_loop(0, EPW // CHUNK, body, 0)
    plsc.subcore_barrier()
    pltpu.sync_copy(acc_sh.at[pl.ds(r0, ROWS_PER_TILE)],
                    out_hbm.at[c, pl.ds(r0, ROWS_PER_TILE)])


@functools.partial(
    pl.kernel,
    out_type=jax.ShapeDtypeStruct((NC, ACC_ROWS, H), jnp.float32),
    mesh=_mesh,
    scratch_types=[
        pltpu.VMEM((CHUNK,), jnp.int32),
        pltpu.VMEM((CHUNK,), jnp.int32),
        pltpu.VMEM((CHUNK, H), jnp.float32),
        pltpu.VMEM_SHARED((ACC_ROWS, H), jnp.float32),
        pltpu.SemaphoreType.DMA,
    ],
)
def _agg_kernel(g0_hbm, g1_hbm, src_hbm, dst_hbm, zeros_hbm, out_hbm,
                src_v, dst_v, rows_v, acc_sh, sem):
    c = lax.axis_index("c")
    s = lax.axis_index("s")
    r0 = s * ROWS_PER_TILE
    pltpu.sync_copy(zeros_hbm.at[pl.ds(r0, ROWS_PER_TILE)],
                    acc_sh.at[pl.ds(r0, ROWS_PER_TILE)])
    plsc.subcore_barrier()

    def body(i, carry):
        base = s * EPT + i * CHUNK
        pltpu.sync_copy(src_hbm.at[pl.ds(base, CHUNK)], src_v)
        pltpu.sync_copy(dst_hbm.at[pl.ds(base, CHUNK)], dst_v)

        @pl.when(c == 0)
        def _():
            pltpu.async_copy(g0_hbm.at[src_v], rows_v, sem).wait()

        @pl.when(c == 1)
        def _():
            pltpu.async_copy(g1_hbm.at[src_v], rows_v, sem).wait()

        pltpu.sync_copy(rows_v, acc_sh.at[dst_v], add=True)
        return carry

    lax.fori_loop(0, NCHUNKS, body, 0)
    plsc.subcore_barrier()
    pltpu.sync_copy(acc_sh.at[pl.ds(r0, ROWS_PER_TILE)],
                    out_hbm.at[c, pl.ds(r0, ROWS_PER_TILE)])


def _leaky(z):
    return jnp.where(z > 0, z, 0.01 * z)


def _prep_body(x_ref, w_ref, dp0_ref, dp1_ref, g0_ref, g1_ref, dinv_ref):
    d = lax.rsqrt(dp0_ref[...] + dp1_ref[...] + 1.0)
    h = jnp.dot(x_ref[...], w_ref[...], preferred_element_type=jnp.float32)
    g = h * d
    g0_ref[...] = g[:, :H]
    g1_ref[...] = g[:, H:]
    dinv_ref[...] = d


_prep_call = pl.pallas_call(
    _prep_body,
    grid=(NBLK,),
    in_specs=[
        pl.BlockSpec((BLK, D), lambda i: (i, 0)),
        pl.BlockSpec((D, D), lambda i: (0, 0)),
        pl.BlockSpec((BLK, 1), lambda i: (i, 0)),
        pl.BlockSpec((BLK, 1), lambda i: (i, 0)),
    ],
    out_specs=[
        pl.BlockSpec((BLK, H), lambda i: (i, 0)),
        pl.BlockSpec((BLK, H), lambda i: (i, 0)),
        pl.BlockSpec((BLK, 1), lambda i: (i, 0)),
    ],
    out_shape=[
        jax.ShapeDtypeStruct((N, H), jnp.float32),
        jax.ShapeDtypeStruct((N, H), jnp.float32),
        jax.ShapeDtypeStruct((N, 1), jnp.float32),
    ],
)


def _mid_body(s0_ref, s1_ref, g0_ref, g1_ref, dinv_ref, b_ref, w_ref,
              o0_ref, o1_ref):
    d = dinv_ref[...]
    b = b_ref[...]
    a0 = _leaky(d * (s0_ref[...] + g0_ref[...]) + b[:, :H])
    a1 = _leaky(d * (s1_ref[...] + g1_ref[...]) + b[:, H:])
    w = w_ref[...]
    h2 = (jnp.dot(a0, w[:H, :], preferred_element_type=jnp.float32)
          + jnp.dot(a1, w[H:, :], preferred_element_type=jnp.float32))
    g2 = h2 * d
    o0_ref[...] = g2[:, :H]
    o1_ref[...] = g2[:, H:]


_mid_call = pl.pallas_call(
    _mid_body,
    grid=(NBLK,),
    in_specs=[
        pl.BlockSpec((BLK, H), lambda i: (i, 0)),
        pl.BlockSpec((BLK, H), lambda i: (i, 0)),
        pl.BlockSpec((BLK, H), lambda i: (i, 0)),
        pl.BlockSpec((BLK, H), lambda i: (i, 0)),
        pl.BlockSpec((BLK, 1), lambda i: (i, 0)),
        pl.BlockSpec((1, D), lambda i: (0, 0)),
        pl.BlockSpec((D, D), lambda i: (0, 0)),
    ],
    out_specs=[
        pl.BlockSpec((BLK, H), lambda i: (i, 0)),
        pl.BlockSpec((BLK, H), lambda i: (i, 0)),
    ],
    out_shape=[
        jax.ShapeDtypeStruct((N, H), jnp.float32),
        jax.ShapeDtypeStruct((N, H), jnp.float32),
    ],
)


def _final_body(s0_ref, s1_ref, g0_ref, g1_ref, dinv_ref, b_ref, o_ref):
    d = dinv_ref[...]
    b = b_ref[...]
    o_ref[:, :H] = _leaky(d * (s0_ref[...] + g0_ref[...]) + b[:, :H])
    o_ref[:, H:] = _leaky(d * (s1_ref[...] + g1_ref[...]) + b[:, H:])


_final_call = pl.pallas_call(
    _final_body,
    grid=(NBLK,),
    in_specs=[
        pl.BlockSpec((BLK, H), lambda i: (i, 0)),
        pl.BlockSpec((BLK, H), lambda i: (i, 0)),
        pl.BlockSpec((BLK, H), lambda i: (i, 0)),
        pl.BlockSpec((BLK, H), lambda i: (i, 0)),
        pl.BlockSpec((BLK, 1), lambda i: (i, 0)),
        pl.BlockSpec((1, D), lambda i: (0, 0)),
    ],
    out_specs=pl.BlockSpec((BLK, D), lambda i: (i, 0)),
    out_shape=jax.ShapeDtypeStruct((N, D), jnp.float32),
)


def kernel(x, edge_index, W1, b1, W2, b2):
    src = edge_index[0].astype(jnp.int32)
    dst = edge_index[1].astype(jnp.int32)
    npad = E_PAD - E
    src_p = jnp.concatenate([src, jnp.zeros((npad,), jnp.int32)])
    dst_p = jnp.concatenate(
        [dst, N + (jnp.arange(npad, dtype=jnp.int32) % (ACC_ROWS - N))])
    zeros_h = jnp.zeros((ACC_ROWS, H), jnp.float32)
    zeros16 = jnp.zeros((ACC_ROWS, 16), jnp.float32)
    ones16 = jnp.ones((CHUNK, 16), jnp.float32)

    degp = _deg_kernel(dst_p, zeros16, ones16)
    dp0 = degp[0, :N, 0:1]
    dp1 = degp[1, :N, 0:1]

    g10, g11, dinv = _prep_call(x, W1, dp0, dp1)
    S1 = _agg_kernel(g10, g11, src_p, dst_p, zeros_h)
    g20, g21 = _mid_call(S1[0, :N], S1[1, :N], g10, g11, dinv,
                         b1.reshape(1, D), W2)
    S2 = _agg_kernel(g20, g21, src_p, dst_p, zeros_h)
    out = _final_call(S2[0, :N], S2[1, :N], g20, g21, dinv,
                      b2.reshape(1, D))
    return out


# speedup vs baseline: 6.1474x; 6.1474x over previous
import functools

import jax
import jax.numpy as jnp
from jax import lax
from jax.experimental import pallas as pl
from jax.experimental.pallas import tpu as pltpu
from jax.experimental.pallas import tpu_sc as plsc

N = 10000
D = 256
H = 128
E = 160000
NC = 2
NS = 16
CHUNK = 128
E_PAD = 163840
EPT = E_PAD // NS
NCHUNKS = EPT // CHUNK
EPW = E_PAD // (NS * NC)
ROWS_PER_TILE = 632
ACC_ROWS = ROWS_PER_TILE * NS
NBLK = 10
BLK = N // NBLK

_mesh = plsc.VectorSubcoreMesh(
    core_axis_name="c", subcore_axis_name="s", num_cores=NC, num_subcores=NS)


def _deg_body(dst_hbm, zeros_hbm, ones_hbm, out_hbm, dst_v, ones_v, acc_sh):
    c = lax.axis_index("c")
    s = lax.axis_index("s")
    wid = s * NC + c
    r0 = s * ROWS_PER_TILE
    pltpu.sync_copy(zeros_hbm.at[pl.ds(r0, ROWS_PER_TILE)],
                    acc_sh.at[pl.ds(r0, ROWS_PER_TILE)])
    pltpu.sync_copy(ones_hbm, ones_v)
    plsc.subcore_barrier()

    def body(i, carry):
        base = wid * EPW + i * CHUNK
        pltpu.sync_copy(dst_hbm.at[pl.ds(base, CHUNK)], dst_v)
        pltpu.sync_copy(ones_v, acc_sh.at[dst_v], add=True)
        return carry

    lax.fori_loop(0, EPW // CHUNK, body, 0)
    plsc.subcore_barrier()
    pltpu.sync_copy(acc_sh.at[pl.ds(r0, ROWS_PER_TILE)],
                    out_hbm.at[c, pl.ds(r0, ROWS_PER_TILE)])


def _make_deg_kernel(interpret=False):
    return functools.partial(
        pl.kernel,
        out_type=jax.ShapeDtypeStruct((NC, ACC_ROWS, H), jnp.float32),
        mesh=_mesh,
        scratch_types=[
            pltpu.VMEM((CHUNK,), jnp.int32),
            pltpu.VMEM((CHUNK, H), jnp.float32),
            pltpu.VMEM_SHARED((ACC_ROWS, H), jnp.float32),
        ],
        interpret=interpret,
    )(_deg_body)


def _agg_body(g_hbm, src_hbm, dst_hbm, zeros_hbm, out_hbm,
              src_v, gi_v, dst_v, rows_v, acc_sh, sem):
    c = lax.axis_index("c")
    s = lax.axis_index("s")
    half0 = c * N
    r0 = s * ROWS_PER_TILE
    pltpu.sync_copy(zeros_hbm.at[pl.ds(r0, ROWS_PER_TILE)],
                    acc_sh.at[pl.ds(r0, ROWS_PER_TILE)])
    plsc.subcore_barrier()

    def body(i, carry):
        base = s * EPT + i * CHUNK
        pltpu.sync_copy(src_hbm.at[pl.ds(base, CHUNK)], src_v)
        pltpu.sync_copy(dst_hbm.at[pl.ds(base, CHUNK)], dst_v)
        for j in range(CHUNK // 16):
            sl = pl.ds(j * 16, 16)
            gi_v[sl] = src_v[sl] + half0
        pltpu.async_copy(g_hbm.at[gi_v], rows_v, sem).wait()
        pltpu.sync_copy(rows_v, acc_sh.at[dst_v], add=True)
        return carry

    lax.fori_loop(0, NCHUNKS, body, 0)
    plsc.subcore_barrier()
    pltpu.sync_copy(acc_sh.at[pl.ds(r0, ROWS_PER_TILE)],
                    out_hbm.at[c, pl.ds(r0, ROWS_PER_TILE)])


def _make_agg_kernel(interpret=False):
    return functools.partial(
        pl.kernel,
        out_type=jax.ShapeDtypeStruct((NC, ACC_ROWS, H), jnp.float32),
        mesh=_mesh,
        scratch_types=[
            pltpu.VMEM((CHUNK,), jnp.int32),
            pltpu.VMEM((CHUNK,), jnp.int32),
            pltpu.VMEM((CHUNK,), jnp.int32),
            pltpu.VMEM((CHUNK, H), jnp.float32),
            pltpu.VMEM_SHARED((ACC_ROWS, H), jnp.float32),
            pltpu.SemaphoreType.DMA,
        ],
        interpret=interpret,
    )(_agg_body)


_deg_kernel = _make_deg_kernel()
_agg_kernel = _make_agg_kernel()


def _leaky(z):
    return jnp.where(z > 0, z, 0.01 * z)


def _prep_body(x_ref, w_ref, dp0_ref, dp1_ref, g_ref, dinv_ref):
    d = lax.rsqrt(dp0_ref[...] + dp1_ref[...] + 1.0)
    h = jnp.dot(x_ref[...], w_ref[...], preferred_element_type=jnp.float32)
    g = h * d
    g_ref[0] = g[:, :H]
    g_ref[1] = g[:, H:]
    dinv_ref[...] = d


_prep_call = pl.pallas_call(
    _prep_body,
    grid=(NBLK,),
    in_specs=[
        pl.BlockSpec((BLK, D), lambda i: (i, 0)),
        pl.BlockSpec((D, D), lambda i: (0, 0)),
        pl.BlockSpec((BLK, 1), lambda i: (i, 0)),
        pl.BlockSpec((BLK, 1), lambda i: (i, 0)),
    ],
    out_specs=[
        pl.BlockSpec((NC, BLK, H), lambda i: (0, i, 0)),
        pl.BlockSpec((BLK, 1), lambda i: (i, 0)),
    ],
    out_shape=[
        jax.ShapeDtypeStruct((NC, N, H), jnp.float32),
        jax.ShapeDtypeStruct((N, 1), jnp.float32),
    ],
)


def _mid_body(s_ref, g_ref, dinv_ref, b_ref, w_ref, o_ref):
    d = dinv_ref[...]
    b = b_ref[...]
    a0 = _leaky(d * (s_ref[0] + g_ref[0]) + b[:, :H])
    a1 = _leaky(d * (s_ref[1] + g_ref[1]) + b[:, H:])
    w = w_ref[...]
    h2 = (jnp.dot(a0, w[:H, :], preferred_element_type=jnp.float32)
          + jnp.dot(a1, w[H:, :], preferred_element_type=jnp.float32))
    g2 = h2 * d
    o_ref[0] = g2[:, :H]
    o_ref[1] = g2[:, H:]


_mid_call = pl.pallas_call(
    _mid_body,
    grid=(NBLK,),
    in_specs=[
        pl.BlockSpec((NC, BLK, H), lambda i: (0, i, 0)),
        pl.BlockSpec((NC, BLK, H), lambda i: (0, i, 0)),
        pl.BlockSpec((BLK, 1), lambda i: (i, 0)),
        pl.BlockSpec((1, D), lambda i: (0, 0)),
        pl.BlockSpec((D, D), lambda i: (0, 0)),
    ],
    out_specs=pl.BlockSpec((NC, BLK, H), lambda i: (0, i, 0)),
    out_shape=jax.ShapeDtypeStruct((NC, N, H), jnp.float32),
)


def _final_body(s_ref, g_ref, dinv_ref, b_ref, o_ref):
    d = dinv_ref[...]
    b = b_ref[...]
    o_ref[:, :H] = _leaky(d * (s_ref[0] + g_ref[0]) + b[:, :H])
    o_ref[:, H:] = _leaky(d * (s_ref[1] + g_ref[1]) + b[:, H:])


_final_call = pl.pallas_call(
    _final_body,
    grid=(NBLK,),
    in_specs=[
        pl.BlockSpec((NC, BLK, H), lambda i: (0, i, 0)),
        pl.BlockSpec((NC, BLK, H), lambda i: (0, i, 0)),
        pl.BlockSpec((BLK, 1), lambda i: (i, 0)),
        pl.BlockSpec((1, D), lambda i: (0, 0)),
    ],
    out_specs=pl.BlockSpec((BLK, D), lambda i: (i, 0)),
    out_shape=jax.ShapeDtypeStruct((N, D), jnp.float32),
)


def kernel(x, edge_index, W1, b1, W2, b2):
    src = edge_index[0].astype(jnp.int32)
    dst = edge_index[1].astype(jnp.int32)
    npad = E_PAD - E
    src_p = jnp.concatenate([src, jnp.zeros((npad,), jnp.int32)])
    dst_p = jnp.concatenate(
        [dst, N + (jnp.arange(npad, dtype=jnp.int32) % (ACC_ROWS - N))])
    zeros_h = jnp.zeros((ACC_ROWS, H), jnp.float32)
    ones_h = jnp.ones((CHUNK, H), jnp.float32)

    degp = _deg_kernel(dst_p, zeros_h, ones_h)
    dp0 = degp[0, :N, 0:1]
    dp1 = degp[1, :N, 0:1]

    g1, dinv = _prep_call(x, W1, dp0, dp1)
    S1 = _agg_kernel(g1.reshape(NC * N, H), src_p, dst_p, zeros_h)
    g2 = _mid_call(S1[:, :N], g1, dinv, b1.reshape(1, D), W2)
    S2 = _agg_kernel(g2.reshape(NC * N, H), src_p, dst_p, zeros_h)
    out = _final_call(S2[:, :N], g2, dinv, b2.reshape(1, D))
    return out


# speedup vs baseline: 7.4748x; 1.2159x over previous
import functools

import jax
import jax.numpy as jnp
from jax import lax
from jax.experimental import pallas as pl
from jax.experimental.pallas import tpu as pltpu
from jax.experimental.pallas import tpu_sc as plsc

N = 10000
D = 256
H = 128
E = 160000
NC = 2
NS = 16
CHUNK = 128
E_PAD = 163840
EPT = E_PAD // NS
NCHUNKS = EPT // CHUNK
EPW = E_PAD // (NS * NC)
ROWS_PER_TILE = 632
ACC_ROWS = ROWS_PER_TILE * NS
NBLK = 10
BLK = N // NBLK

_mesh = plsc.VectorSubcoreMesh(
    core_axis_name="c", subcore_axis_name="s", num_cores=NC, num_subcores=NS)


def _deg_body(dst_hbm, zeros_hbm, ones_hbm, out_hbm, dst_v, ones_v, acc_sh):
    c = lax.axis_index("c")
    s = lax.axis_index("s")
    wid = s * NC + c
    r0 = s * ROWS_PER_TILE
    pltpu.sync_copy(zeros_hbm.at[pl.ds(r0, ROWS_PER_TILE)],
                    acc_sh.at[pl.ds(r0, ROWS_PER_TILE)])
    pltpu.sync_copy(ones_hbm, ones_v)
    plsc.subcore_barrier()

    def body(i, carry):
        base = wid * EPW + i * CHUNK
        pltpu.sync_copy(dst_hbm.at[pl.ds(base, CHUNK)], dst_v)
        pltpu.sync_copy(ones_v, acc_sh.at[dst_v], add=True)
        return carry

    lax.fori_loop(0, EPW // CHUNK, body, 0)
    plsc.subcore_barrier()
    pltpu.sync_copy(acc_sh.at[pl.ds(r0, ROWS_PER_TILE)],
                    out_hbm.at[c, pl.ds(r0, ROWS_PER_TILE)])


def _make_deg_kernel(interpret=False):
    return functools.partial(
        pl.kernel,
        out_type=jax.ShapeDtypeStruct((NC, ACC_ROWS, H), jnp.float32),
        mesh=_mesh,
        scratch_types=[
            pltpu.VMEM((CHUNK,), jnp.int32),
            pltpu.VMEM((CHUNK, H), jnp.float32),
            pltpu.VMEM_SHARED((ACC_ROWS, H), jnp.float32),
        ],
        interpret=interpret,
    )(_deg_body)


SBLK = 16
NSB = NCHUNKS // SBLK


def _agg_body(g_hbm, src_hbm, dst_hbm, zeros_hbm, out_hbm,
              sg_v, dst_v, rows0, rows1, acc_sh, gs0, gs1, ss0, ss1):
    c = lax.axis_index("c")
    s = lax.axis_index("s")
    r0 = s * ROWS_PER_TILE
    pltpu.sync_copy(zeros_hbm.at[pl.ds(r0, ROWS_PER_TILE)],
                    acc_sh.at[pl.ds(r0, ROWS_PER_TILE)])
    row0 = s * NCHUNKS
    pltpu.sync_copy(dst_hbm.at[pl.ds(row0, NCHUNKS)], dst_v)
    half0 = c * N
    plsc.subcore_barrier()

    def g_start(k, i, buf, sem):
        del i
        pltpu.async_copy(g_hbm.at[sg_v.at[k]], buf, sem)

    def g_wait(k, i, buf, sem):
        del i
        pltpu.make_async_copy(g_hbm.at[sg_v.at[k]], buf, sem).wait()

    def s_start(i, buf, sem):
        pltpu.async_copy(buf, acc_sh.at[dst_v.at[i]], sem, add=True)

    def s_wait(i, buf, sem):
        pltpu.make_async_copy(buf, acc_sh.at[dst_v.at[i]], sem).wait()

    def superblock(sb, carry):
        pltpu.sync_copy(src_hbm.at[pl.ds(row0 + sb * SBLK, SBLK)], sg_v)

        def fill(k, carry):
            for j in range(CHUNK // 16):
                sl = pl.ds(j * 16, 16)
                sg_v[k, sl] = sg_v[k, sl] + half0
            return carry

        lax.fori_loop(0, SBLK, fill, 0)

        base = sb * SBLK
        g_start(0, base, rows0, gs0)

        def pipe(k2, carry):
            k0 = k2 * 2
            k1 = k0 + 1
            g_wait(k0, base + k0, rows0, gs0)
            s_start(base + k0, rows0, ss0)

            @pl.when(k2 > 0)
            def _():
                s_wait(base + k1 - 2, rows1, ss1)

            g_start(k1, base + k1, rows1, gs1)
            g_wait(k1, base + k1, rows1, gs1)
            s_start(base + k1, rows1, ss1)
            s_wait(base + k0, rows0, ss0)

            @pl.when(k2 < SBLK // 2 - 1)
            def _():
                g_start(k0 + 2, base + k0 + 2, rows0, gs0)

            return carry

        lax.fori_loop(0, SBLK // 2, pipe, 0)
        s_wait(base + SBLK - 1, rows1, ss1)
        return carry

    lax.fori_loop(0, NSB, superblock, 0)
    plsc.subcore_barrier()
    pltpu.sync_copy(acc_sh.at[pl.ds(r0, ROWS_PER_TILE)],
                    out_hbm.at[c, pl.ds(r0, ROWS_PER_TILE)])


def _make_agg_kernel(interpret=False):
    return functools.partial(
        pl.kernel,
        out_type=jax.ShapeDtypeStruct((NC, ACC_ROWS, H), jnp.float32),
        mesh=_mesh,
        scratch_types=[
            pltpu.VMEM((SBLK, CHUNK), jnp.int32),
            pltpu.VMEM((NCHUNKS, CHUNK), jnp.int32),
            pltpu.VMEM((CHUNK, H), jnp.float32),
            pltpu.VMEM((CHUNK, H), jnp.float32),
            pltpu.VMEM_SHARED((ACC_ROWS, H), jnp.float32),
            pltpu.SemaphoreType.DMA,
            pltpu.SemaphoreType.DMA,
            pltpu.SemaphoreType.DMA,
            pltpu.SemaphoreType.DMA,
        ],
        interpret=interpret,
    )(_agg_body)


_deg_kernel = _make_deg_kernel()
_agg_kernel = _make_agg_kernel()


def _leaky(z):
    return jnp.where(z > 0, z, 0.01 * z)


def _prep_body(x_ref, w_ref, dp0_ref, dp1_ref, g_ref, dinv_ref):
    d = lax.rsqrt(dp0_ref[...] + dp1_ref[...] + 1.0)
    h = jnp.dot(x_ref[...], w_ref[...], preferred_element_type=jnp.float32)
    g = h * d
    g_ref[0] = g[:, :H]
    g_ref[1] = g[:, H:]
    dinv_ref[...] = d


_prep_call = pl.pallas_call(
    _prep_body,
    grid=(NBLK,),
    in_specs=[
        pl.BlockSpec((BLK, D), lambda i: (i, 0)),
        pl.BlockSpec((D, D), lambda i: (0, 0)),
        pl.BlockSpec((BLK, 1), lambda i: (i, 0)),
        pl.BlockSpec((BLK, 1), lambda i: (i, 0)),
    ],
    out_specs=[
        pl.BlockSpec((NC, BLK, H), lambda i: (0, i, 0)),
        pl.BlockSpec((BLK, 1), lambda i: (i, 0)),
    ],
    out_shape=[
        jax.ShapeDtypeStruct((NC, N, H), jnp.float32),
        jax.ShapeDtypeStruct((N, 1), jnp.float32),
    ],
)


def _mid_body(s_ref, g_ref, dinv_ref, b_ref, w_ref, o_ref):
    d = dinv_ref[...]
    b = b_ref[...]
    a0 = _leaky(d * (s_ref[0] + g_ref[0]) + b[:, :H])
    a1 = _leaky(d * (s_ref[1] + g_ref[1]) + b[:, H:])
    w = w_ref[...]
    h2 = (jnp.dot(a0, w[:H, :], preferred_element_type=jnp.float32)
          + jnp.dot(a1, w[H:, :], preferred_element_type=jnp.float32))
    g2 = h2 * d
    o_ref[0] = g2[:, :H]
    o_ref[1] = g2[:, H:]


_mid_call = pl.pallas_call(
    _mid_body,
    grid=(NBLK,),
    in_specs=[
        pl.BlockSpec((NC, BLK, H), lambda i: (0, i, 0)),
        pl.BlockSpec((NC, BLK, H), lambda i: (0, i, 0)),
        pl.BlockSpec((BLK, 1), lambda i: (i, 0)),
        pl.BlockSpec((1, D), lambda i: (0, 0)),
        pl.BlockSpec((D, D), lambda i: (0, 0)),
    ],
    out_specs=pl.BlockSpec((NC, BLK, H), lambda i: (0, i, 0)),
    out_shape=jax.ShapeDtypeStruct((NC, N, H), jnp.float32),
)


def _final_body(s_ref, g_ref, dinv_ref, b_ref, o_ref):
    d = dinv_ref[...]
    b = b_ref[...]
    o_ref[:, :H] = _leaky(d * (s_ref[0] + g_ref[0]) + b[:, :H])
    o_ref[:, H:] = _leaky(d * (s_ref[1] + g_ref[1]) + b[:, H:])


_final_call = pl.pallas_call(
    _final_body,
    grid=(NBLK,),
    in_specs=[
        pl.BlockSpec((NC, BLK, H), lambda i: (0, i, 0)),
        pl.BlockSpec((NC, BLK, H), lambda i: (0, i, 0)),
        pl.BlockSpec((BLK, 1), lambda i: (i, 0)),
        pl.BlockSpec((1, D), lambda i: (0, 0)),
    ],
    out_specs=pl.BlockSpec((BLK, D), lambda i: (i, 0)),
    out_shape=jax.ShapeDtypeStruct((N, D), jnp.float32),
)


def kernel(x, edge_index, W1, b1, W2, b2):
    src = edge_index[0].astype(jnp.int32)
    dst = edge_index[1].astype(jnp.int32)
    npad = E_PAD - E
    src_p = jnp.concatenate([src, jnp.zeros((npad,), jnp.int32)])
    dst_p = jnp.concatenate(
        [dst, N + (jnp.arange(npad, dtype=jnp.int32) % (ACC_ROWS - N))])
    zeros_h = jnp.zeros((ACC_ROWS, H), jnp.float32)
    ones_h = jnp.ones((CHUNK, H), jnp.float32)

    degp = _deg_kernel(dst_p, zeros_h, ones_h)
    dp0 = degp[0, :N, 0:1]
    dp1 = degp[1, :N, 0:1]

    src2d = src_p.reshape(E_PAD // CHUNK, CHUNK)
    dst2d = dst_p.reshape(E_PAD // CHUNK, CHUNK)
    g1, dinv = _prep_call(x, W1, dp0, dp1)
    S1 = _agg_kernel(g1.reshape(NC * N, H), src2d, dst2d, zeros_h)
    g2 = _mid_call(S1[:, :N], g1, dinv, b1.reshape(1, D), W2)
    S2 = _agg_kernel(g2.reshape(NC * N, H), src2d, dst2d, zeros_h)
    out = _final_call(S2[:, :N], g2, dinv, b2.reshape(1, D))
    return out


# speedup vs baseline: 7.9949x; 1.0696x over previous
import functools

import jax
import jax.numpy as jnp
from jax import lax
from jax.experimental import pallas as pl
from jax.experimental.pallas import tpu as pltpu
from jax.experimental.pallas import tpu_sc as plsc

N = 10000
D = 256
H = 128
E = 160000
NC = 2
NS = 16
CHUNK = 128
E_PAD = 163840
EPT = E_PAD // NS
NCHUNKS = EPT // CHUNK
EPW = E_PAD // (NS * NC)
ROWS_PER_TILE = 640
ACC_ROWS = ROWS_PER_TILE * NS
DEG_CHUNKS = EPW // CHUNK
NBLK = 10
BLK = N // NBLK

_mesh = plsc.VectorSubcoreMesh(
    core_axis_name="c", subcore_axis_name="s", num_cores=NC, num_subcores=NS)


def _deg_body(dst_hbm, zeros_hbm, ones_hbm, out_hbm, dst_v, ones_v, acc_sh,
              sem):
    c = lax.axis_index("c")
    s = lax.axis_index("s")
    wid = s * NC + c
    r0 = s * ROWS_PER_TILE
    pltpu.sync_copy(zeros_hbm.at[pl.ds(r0, ROWS_PER_TILE)],
                    acc_sh.at[pl.ds(r0, ROWS_PER_TILE)])
    pltpu.sync_copy(ones_hbm, ones_v)
    pltpu.sync_copy(dst_hbm.at[pl.ds(wid * DEG_CHUNKS, DEG_CHUNKS)], dst_v)
    plsc.subcore_barrier()

    def fire(i, carry):
        pltpu.async_copy(ones_v, acc_sh.at[dst_v.at[i]], sem, add=True)

        @pl.when(i >= 2)
        def _():
            pltpu.make_async_copy(ones_v, acc_sh.at[dst_v.at[i - 2]],
                                  sem).wait()

        return carry

    lax.fori_loop(0, DEG_CHUNKS, fire, 0)

    def drain(i, carry):
        pltpu.make_async_copy(ones_v, acc_sh.at[dst_v.at[i]], sem).wait()
        return carry

    lax.fori_loop(DEG_CHUNKS - 2, DEG_CHUNKS, drain, 0)
    plsc.subcore_barrier()
    pltpu.sync_copy(acc_sh.at[pl.ds(r0, ROWS_PER_TILE)],
                    out_hbm.at[c, pl.ds(r0, ROWS_PER_TILE)])


def _make_deg_kernel(interpret=False):
    return functools.partial(
        pl.kernel,
        out_type=jax.ShapeDtypeStruct((NC, ACC_ROWS, H), jnp.float32),
        mesh=_mesh,
        scratch_types=[
            pltpu.VMEM((DEG_CHUNKS, CHUNK), jnp.int32),
            pltpu.VMEM((CHUNK, H), jnp.float32),
            pltpu.VMEM_SHARED((ACC_ROWS, H), jnp.float32),
            pltpu.SemaphoreType.DMA,
        ],
        interpret=interpret,
    )(_deg_body)


SBLK = 16
NSB = NCHUNKS // SBLK


def _agg_body(g_hbm, src_hbm, dst_hbm, zeros_hbm, out_hbm,
              sg_v, dst_v, rows0, rows1, acc_sh, gs0, gs1, ss0, ss1):
    c = lax.axis_index("c")
    s = lax.axis_index("s")
    r0 = s * ROWS_PER_TILE
    pltpu.sync_copy(zeros_hbm.at[pl.ds(r0, ROWS_PER_TILE)],
                    acc_sh.at[pl.ds(r0, ROWS_PER_TILE)])
    row0 = s * NCHUNKS
    pltpu.sync_copy(dst_hbm.at[pl.ds(row0, NCHUNKS)], dst_v)
    half0 = c * N
    plsc.subcore_barrier()

    def g_start(k, i, buf, sem):
        del i
        pltpu.async_copy(g_hbm.at[sg_v.at[k]], buf, sem)

    def g_wait(k, i, buf, sem):
        del i
        pltpu.make_async_copy(g_hbm.at[sg_v.at[k]], buf, sem).wait()

    def s_start(i, buf, sem):
        pltpu.async_copy(buf, acc_sh.at[dst_v.at[i]], sem, add=True)

    def s_wait(i, buf, sem):
        pltpu.make_async_copy(buf, acc_sh.at[dst_v.at[i]], sem).wait()

    def superblock(sb, carry):
        pltpu.sync_copy(src_hbm.at[pl.ds(row0 + sb * SBLK, SBLK)], sg_v)

        def fill(k, carry):
            for j in range(CHUNK // 16):
                sl = pl.ds(j * 16, 16)
                sg_v[k, sl] = sg_v[k, sl] + half0
            return carry

        lax.fori_loop(0, SBLK, fill, 0)

        base = sb * SBLK
        g_start(0, base, rows0, gs0)

        def pipe(k2, carry):
            k0 = k2 * 2
            k1 = k0 + 1
            g_wait(k0, base + k0, rows0, gs0)
            s_start(base + k0, rows0, ss0)

            @pl.when(k2 > 0)
            def _():
                s_wait(base + k1 - 2, rows1, ss1)

            g_start(k1, base + k1, rows1, gs1)
            g_wait(k1, base + k1, rows1, gs1)
            s_start(base + k1, rows1, ss1)
            s_wait(base + k0, rows0, ss0)

            @pl.when(k2 < SBLK // 2 - 1)
            def _():
                g_start(k0 + 2, base + k0 + 2, rows0, gs0)

            return carry

        lax.fori_loop(0, SBLK // 2, pipe, 0)
        s_wait(base + SBLK - 1, rows1, ss1)
        return carry

    lax.fori_loop(0, NSB, superblock, 0)
    plsc.subcore_barrier()
    pltpu.sync_copy(acc_sh.at[pl.ds(r0, ROWS_PER_TILE)],
                    out_hbm.at[c, pl.ds(r0, ROWS_PER_TILE)])


def _make_agg_kernel(interpret=False):
    return functools.partial(
        pl.kernel,
        out_type=jax.ShapeDtypeStruct((NC, ACC_ROWS, H), jnp.float32),
        mesh=_mesh,
        scratch_types=[
            pltpu.VMEM((SBLK, CHUNK), jnp.int32),
            pltpu.VMEM((NCHUNKS, CHUNK), jnp.int32),
            pltpu.VMEM((CHUNK, H), jnp.float32),
            pltpu.VMEM((CHUNK, H), jnp.float32),
            pltpu.VMEM_SHARED((ACC_ROWS, H), jnp.float32),
            pltpu.SemaphoreType.DMA,
            pltpu.SemaphoreType.DMA,
            pltpu.SemaphoreType.DMA,
            pltpu.SemaphoreType.DMA,
        ],
        interpret=interpret,
    )(_agg_body)


_deg_kernel = _make_deg_kernel()
_agg_kernel = _make_agg_kernel()


def _leaky(z):
    return jnp.where(z > 0, z, 0.01 * z)


def _prep_body(x_ref, w_ref, dp0_ref, dp1_ref, g_ref, dinv_ref):
    d = lax.rsqrt(dp0_ref[...] + dp1_ref[...] + 1.0)
    h = jnp.dot(x_ref[...], w_ref[...], preferred_element_type=jnp.float32)
    g = h * d
    g_ref[0] = g[:, :H]
    g_ref[1] = g[:, H:]
    dinv_ref[...] = d


_prep_call = pl.pallas_call(
    _prep_body,
    grid=(NBLK,),
    in_specs=[
        pl.BlockSpec((BLK, D), lambda i: (i, 0)),
        pl.BlockSpec((D, D), lambda i: (0, 0)),
        pl.BlockSpec((BLK, 1), lambda i: (i, 0)),
        pl.BlockSpec((BLK, 1), lambda i: (i, 0)),
    ],
    out_specs=[
        pl.BlockSpec((NC, BLK, H), lambda i: (0, i, 0)),
        pl.BlockSpec((BLK, 1), lambda i: (i, 0)),
    ],
    out_shape=[
        jax.ShapeDtypeStruct((NC, N, H), jnp.float32),
        jax.ShapeDtypeStruct((N, 1), jnp.float32),
    ],
)


def _mid_body(s_ref, g_ref, dinv_ref, b_ref, w_ref, o_ref):
    d = dinv_ref[...]
    b = b_ref[...]
    a0 = _leaky(d * (s_ref[0] + g_ref[0]) + b[:, :H])
    a1 = _leaky(d * (s_ref[1] + g_ref[1]) + b[:, H:])
    w = w_ref[...]
    h2 = (jnp.dot(a0, w[:H, :], preferred_element_type=jnp.float32)
          + jnp.dot(a1, w[H:, :], preferred_element_type=jnp.float32))
    g2 = h2 * d
    o_ref[0] = g2[:, :H]
    o_ref[1] = g2[:, H:]


_mid_call = pl.pallas_call(
    _mid_body,
    grid=(NBLK,),
    in_specs=[
        pl.BlockSpec((NC, BLK, H), lambda i: (0, i, 0)),
        pl.BlockSpec((NC, BLK, H), lambda i: (0, i, 0)),
        pl.BlockSpec((BLK, 1), lambda i: (i, 0)),
        pl.BlockSpec((1, D), lambda i: (0, 0)),
        pl.BlockSpec((D, D), lambda i: (0, 0)),
    ],
    out_specs=pl.BlockSpec((NC, BLK, H), lambda i: (0, i, 0)),
    out_shape=jax.ShapeDtypeStruct((NC, N, H), jnp.float32),
)


def _final_body(s_ref, g_ref, dinv_ref, b_ref, o_ref):
    d = dinv_ref[...]
    b = b_ref[...]
    o_ref[:, :H] = _leaky(d * (s_ref[0] + g_ref[0]) + b[:, :H])
    o_ref[:, H:] = _leaky(d * (s_ref[1] + g_ref[1]) + b[:, H:])


_final_call = pl.pallas_call(
    _final_body,
    grid=(NBLK,),
    in_specs=[
        pl.BlockSpec((NC, BLK, H), lambda i: (0, i, 0)),
        pl.BlockSpec((NC, BLK, H), lambda i: (0, i, 0)),
        pl.BlockSpec((BLK, 1), lambda i: (i, 0)),
        pl.BlockSpec((1, D), lambda i: (0, 0)),
    ],
    out_specs=pl.BlockSpec((BLK, D), lambda i: (i, 0)),
    out_shape=jax.ShapeDtypeStruct((N, D), jnp.float32),
)


def kernel(x, edge_index, W1, b1, W2, b2):
    src = edge_index[0].astype(jnp.int32)
    dst = edge_index[1].astype(jnp.int32)
    npad = E_PAD - E
    src_p = jnp.concatenate([src, jnp.zeros((npad,), jnp.int32)])
    dst_p = jnp.concatenate(
        [dst, N + (jnp.arange(npad, dtype=jnp.int32) % (ACC_ROWS - N))])
    zeros_h = jnp.zeros((ACC_ROWS, H), jnp.float32)
    src2d = src_p.reshape(E_PAD // CHUNK, CHUNK)
    dst2d = dst_p.reshape(E_PAD // CHUNK, CHUNK)

    ones_h = jnp.ones((CHUNK, H), jnp.float32)
    degp = _deg_kernel(dst2d, zeros_h, ones_h)
    dp0 = degp[0, :N, 0:1]
    dp1 = degp[1, :N, 0:1]

    g1, dinv = _prep_call(x, W1, dp0, dp1)
    S1 = _agg_kernel(g1.reshape(NC * N, H), src2d, dst2d, zeros_h)
    g2 = _mid_call(S1, g1, dinv, b1.reshape(1, D), W2)
    S2 = _agg_kernel(g2.reshape(NC * N, H), src2d, dst2d, zeros_h)
    out = _final_call(S2, g2, dinv, b2.reshape(1, D))
    return out
